# baseline (speedup 1.0000x reference)
import functools

import jax
import jax.numpy as jnp
from jax import lax
from jax.experimental import pallas as pl
from jax.experimental.pallas import tpu as pltpu
from jax.experimental.pallas import tpu_sc as plsc

_BATCH = 4096
_HIST = 200
_HIDDEN = 64
_ITEMS = 1000000
_G = 2
_SPLITS = ((0, 96), (96, 104))
_TPR = 2048
_NMAIN = (_ITEMS // _TPR) * _TPR
_NTAIL = _ITEMS - _NMAIN


def _tp_body(tt_ref, tail_ref, out_ref):
    x = tt_ref[...]
    out_ref[:, :_HIDDEN] = x.T

    @pl.when(pl.program_id(0) == 0)
    def _():
        out_ref[:_NTAIL, _HIDDEN:] = tail_ref[...].T


_tp = pl.pallas_call(
    _tp_body,
    grid=(_ITEMS // _TPR,),
    in_specs=[
        pl.BlockSpec((_HIDDEN, _TPR), lambda i: (0, i)),
        pl.BlockSpec((_HIDDEN, _NTAIL), lambda i: (0, 0)),
    ],
    out_specs=pl.BlockSpec((_TPR, 2 * _HIDDEN), lambda i: (i, 0)),
    out_shape=jax.ShapeDtypeStruct((_NMAIN, 2 * _HIDDEN), jnp.float32),
)


def _make_gather():
    info = plsc.get_sparse_core_info()
    nw = info.num_cores * info.num_subcores
    rows_per_w = _BATCH // nw
    n_steps = rows_per_w // _G
    mesh = plsc.VectorSubcoreMesh(core_axis_name="c", subcore_axis_name="s")

    @functools.partial(
        pl.kernel,
        mesh=mesh,
        out_type=jax.ShapeDtypeStruct((_BATCH, _HIST, 2 * _HIDDEN), jnp.float32),
        scratch_types=[
            pltpu.VMEM((rows_per_w, _HIST), jnp.int32),
            pltpu.VMEM((_G, _HIST, _HIDDEN), jnp.float32),
            pltpu.VMEM((_G, _HIST, _HIDDEN), jnp.float32),
            pltpu.SemaphoreType.DMA,
            pltpu.SemaphoreType.DMA,
            pltpu.SemaphoreType.DMA,
            pltpu.SemaphoreType.DMA,
        ],
        compiler_params=pltpu.CompilerParams(use_tc_tiling_on_sc=False),
    )
    def gather_kernel(idx_hbm, table_hbm, out_hbm, idx_v, rows0, rows1,
                      si0, si1, so0, so1):
        wid = lax.axis_index("s") * info.num_cores + lax.axis_index("c")
        base = wid * rows_per_w
        pltpu.sync_copy(idx_hbm.at[pl.ds(base, rows_per_w)], idx_v)

        def fire(c, rows, sem):
            for j in range(_G):
                for off, width in _SPLITS:
                    pltpu.async_copy(
                        table_hbm.at[idx_v.at[_G * c + j, pl.ds(off, width)]],
                        rows.at[j, pl.ds(off, width)],
                        sem,
                    )

        def drain(c, rows, sem):
            for j in range(_G):
                for off, width in _SPLITS:
                    pltpu.make_async_copy(
                        table_hbm.at[idx_v.at[_G * c + j, pl.ds(off, width)]],
                        rows.at[j, pl.ds(off, width)],
                        sem,
                    ).wait()

        def write(c, rows, sem):
            return pltpu.async_copy(
                rows,
                out_hbm.at[pl.ds(base + _G * c, _G), :, pl.ds(0, _HIDDEN)],
                sem)

        def wait_write(c, rows, sem):
            pltpu.make_async_copy(
                rows,
                out_hbm.at[pl.ds(base + _G * c, _G), :, pl.ds(0, _HIDDEN)],
                sem).wait()

        bufs = ((rows0, si0, so0), (rows1, si1, so1))

        def body(cc, carry):
            for b in range(2):
                rows, si, so = bufs[b]
                o_rows, o_si, o_so = bufs[1 - b]
                c = 2 * cc + b

                @pl.when(c >= 2)
                def _():
                    wait_write(c - 2, rows, so)

                fire(c, rows, si)

                @pl.when(c >= 1)
                def _():
                    drain(c - 1, o_rows, o_si)
                    write(c - 1, o_rows, o_so)

            return carry

        lax.fori_loop(0, n_steps // 2, body, 0)

        last = n_steps - 1
        rows, si, so = bufs[last % 2]
        o_rows, o_si, o_so = bufs[1 - last % 2]
        drain(last, rows, si)
        write(last, rows, so)
        wait_write(last - 1, o_rows, o_so)
        wait_write(last, rows, so)

    return gather_kernel


_gather = _make_gather()


def kernel(item_ids, table):
    ids = item_ids.astype(jnp.int32)
    idx2 = jnp.where(ids < _NMAIN, 2 * ids, 2 * (ids - _NMAIN) + 1)
    tail_t = table.T[:, _NMAIN:]
    tpad = _tp(table.T, tail_t)
    t2 = tpad.reshape(2 * _NMAIN, _HIDDEN)
    outp = _gather(idx2, t2)
    return outp[..., :_HIDDEN]


# speedup vs baseline: 1.3264x; 1.3264x over previous
import functools

import jax
import jax.numpy as jnp
from jax import lax
from jax.experimental import pallas as pl
from jax.experimental.pallas import tpu as pltpu
from jax.experimental.pallas import tpu_sc as plsc

_BATCH = 4096
_HIST = 200
_HIDDEN = 64
_ITEMS = 1000000
_G = 2
_SPLITS = ((0, 96), (96, 104))
_TPR = 8192
_NMAIN = (_ITEMS // _TPR) * _TPR
_NTAIL = _ITEMS - _NMAIN


def _tp_body(tt_ref, tail_ref, out_ref):
    x = tt_ref[...]
    out_ref[:, :_HIDDEN] = x.T

    @pl.when(pl.program_id(0) == 0)
    def _():
        out_ref[:_NTAIL, _HIDDEN:] = tail_ref[...].T


_tp = pl.pallas_call(
    _tp_body,
    grid=(_ITEMS // _TPR,),
    in_specs=[
        pl.BlockSpec((_HIDDEN, _TPR), lambda i: (0, i)),
        pl.BlockSpec((_HIDDEN, _NTAIL), lambda i: (0, 0)),
    ],
    out_specs=pl.BlockSpec((_TPR, 2 * _HIDDEN), lambda i: (i, 0)),
    out_shape=jax.ShapeDtypeStruct((_NMAIN, 2 * _HIDDEN), jnp.float32),
)


def _make_gather():
    info = plsc.get_sparse_core_info()
    nw = info.num_cores * info.num_subcores
    rows_per_w = _BATCH // nw
    n_steps = rows_per_w // _G
    mesh = plsc.VectorSubcoreMesh(core_axis_name="c", subcore_axis_name="s")

    @functools.partial(
        pl.kernel,
        mesh=mesh,
        out_type=jax.ShapeDtypeStruct((_BATCH, _HIST, 2 * _HIDDEN), jnp.float32),
        scratch_types=[
            pltpu.VMEM((rows_per_w, _HIST), jnp.int32),
            pltpu.VMEM((_G, _HIST, _HIDDEN), jnp.float32),
            pltpu.VMEM((_G, _HIST, _HIDDEN), jnp.float32),
            pltpu.SemaphoreType.DMA,
            pltpu.SemaphoreType.DMA,
            pltpu.SemaphoreType.DMA,
            pltpu.SemaphoreType.DMA,
        ],
        compiler_params=pltpu.CompilerParams(use_tc_tiling_on_sc=False),
    )
    def gather_kernel(idx_hbm, table_hbm, out_hbm, idx_v, rows0, rows1,
                      si0, si1, so0, so1):
        wid = lax.axis_index("s") * info.num_cores + lax.axis_index("c")
        base = wid * rows_per_w
        pltpu.sync_copy(idx_hbm.at[pl.ds(base, rows_per_w)], idx_v)

        def fire(c, rows, sem):
            for j in range(_G):
                for off, width in _SPLITS:
                    pltpu.async_copy(
                        table_hbm.at[idx_v.at[_G * c + j, pl.ds(off, width)]],
                        rows.at[j, pl.ds(off, width)],
                        sem,
                    )

        def drain(c, rows, sem):
            for j in range(_G):
                for off, width in _SPLITS:
                    pltpu.make_async_copy(
                        table_hbm.at[idx_v.at[_G * c + j, pl.ds(off, width)]],
                        rows.at[j, pl.ds(off, width)],
                        sem,
                    ).wait()

        def write(c, rows, sem):
            return pltpu.async_copy(
                rows,
                out_hbm.at[pl.ds(base + _G * c, _G), :, pl.ds(0, _HIDDEN)],
                sem)

        def wait_write(c, rows, sem):
            pltpu.make_async_copy(
                rows,
                out_hbm.at[pl.ds(base + _G * c, _G), :, pl.ds(0, _HIDDEN)],
                sem).wait()

        bufs = ((rows0, si0, so0), (rows1, si1, so1))

        def body(cc, carry):
            for b in range(2):
                rows, si, so = bufs[b]
                o_rows, o_si, o_so = bufs[1 - b]
                c = 2 * cc + b

                @pl.when(c >= 2)
                def _():
                    wait_write(c - 2, rows, so)

                fire(c, rows, si)

                @pl.when(c >= 1)
                def _():
                    drain(c - 1, o_rows, o_si)
                    write(c - 1, o_rows, o_so)

            return carry

        lax.fori_loop(0, n_steps // 2, body, 0)

        last = n_steps - 1
        rows, si, so = bufs[last % 2]
        o_rows, o_si, o_so = bufs[1 - last % 2]
        drain(last, rows, si)
        write(last, rows, so)
        wait_write(last - 1, o_rows, o_so)
        wait_write(last, rows, so)

    return gather_kernel


_gather = _make_gather()


def kernel(item_ids, table):
    ids = item_ids.astype(jnp.int32)
    idx2 = jnp.where(ids < _NMAIN, 2 * ids, 2 * (ids - _NMAIN) + 1)
    tail_t = table.T[:, _NMAIN:]
    tpad = _tp(table.T, tail_t)
    t2 = tpad.reshape(2 * _NMAIN, _HIDDEN)
    outp = _gather(idx2, t2)
    return outp[..., :_HIDDEN]


# speedup vs baseline: 1.3664x; 1.0301x over previous
import functools

import jax
import jax.numpy as jnp
from jax import lax
from jax.experimental import pallas as pl
from jax.experimental.pallas import tpu as pltpu
from jax.experimental.pallas import tpu_sc as plsc

_BATCH = 4096
_HIST = 200
_HIDDEN = 64
_ITEMS = 1000000
_G = 2
_SPLITS = ((0, 96), (96, 104))
_TPR = 16384
_NMAIN = (_ITEMS // _TPR) * _TPR
_NTAIL = _ITEMS - _NMAIN


def _tp_body(tt_ref, tail_ref, out_ref):
    x = tt_ref[...]
    out_ref[:, :_HIDDEN] = x.T

    @pl.when(pl.program_id(0) == 0)
    def _():
        out_ref[:_NTAIL, _HIDDEN:] = tail_ref[...].T


_tp = pl.pallas_call(
    _tp_body,
    grid=(_ITEMS // _TPR,),
    in_specs=[
        pl.BlockSpec((_HIDDEN, _TPR), lambda i: (0, i)),
        pl.BlockSpec((_HIDDEN, _NTAIL), lambda i: (0, 0)),
    ],
    out_specs=pl.BlockSpec((_TPR, 2 * _HIDDEN), lambda i: (i, 0)),
    out_shape=jax.ShapeDtypeStruct((_NMAIN, 2 * _HIDDEN), jnp.float32),
)


def _make_gather():
    info = plsc.get_sparse_core_info()
    nw = info.num_cores * info.num_subcores
    rows_per_w = _BATCH // nw
    n_steps = rows_per_w // _G
    mesh = plsc.VectorSubcoreMesh(core_axis_name="c", subcore_axis_name="s")

    @functools.partial(
        pl.kernel,
        mesh=mesh,
        out_type=jax.ShapeDtypeStruct((_BATCH, _HIST, 2 * _HIDDEN), jnp.float32),
        scratch_types=[
            pltpu.VMEM((rows_per_w, _HIST), jnp.int32),
            pltpu.VMEM((_G, _HIST, _HIDDEN), jnp.float32),
            pltpu.VMEM((_G, _HIST, _HIDDEN), jnp.float32),
            pltpu.SemaphoreType.DMA,
            pltpu.SemaphoreType.DMA,
            pltpu.SemaphoreType.DMA,
            pltpu.SemaphoreType.DMA,
        ],
        compiler_params=pltpu.CompilerParams(use_tc_tiling_on_sc=False),
    )
    def gather_kernel(idx_hbm, table_hbm, out_hbm, idx_v, rows0, rows1,
                      si0, si1, so0, so1):
        wid = lax.axis_index("s") * info.num_cores + lax.axis_index("c")
        base = wid * rows_per_w
        pltpu.sync_copy(idx_hbm.at[pl.ds(base, rows_per_w)], idx_v)

        def fire(c, rows, sem):
            for j in range(_G):
                for off, width in _SPLITS:
                    pltpu.async_copy(
                        table_hbm.at[idx_v.at[_G * c + j, pl.ds(off, width)]],
                        rows.at[j, pl.ds(off, width)],
                        sem,
                    )

        def drain(c, rows, sem):
            for j in range(_G):
                for off, width in _SPLITS:
                    pltpu.make_async_copy(
                        table_hbm.at[idx_v.at[_G * c + j, pl.ds(off, width)]],
                        rows.at[j, pl.ds(off, width)],
                        sem,
                    ).wait()

        def write(c, rows, sem):
            return pltpu.async_copy(
                rows,
                out_hbm.at[pl.ds(base + _G * c, _G), :, pl.ds(0, _HIDDEN)],
                sem)

        def wait_write(c, rows, sem):
            pltpu.make_async_copy(
                rows,
                out_hbm.at[pl.ds(base + _G * c, _G), :, pl.ds(0, _HIDDEN)],
                sem).wait()

        bufs = ((rows0, si0, so0), (rows1, si1, so1))

        def body(cc, carry):
            for b in range(2):
                rows, si, so = bufs[b]
                o_rows, o_si, o_so = bufs[1 - b]
                c = 2 * cc + b

                @pl.when(c >= 2)
                def _():
                    wait_write(c - 2, rows, so)

                fire(c, rows, si)

                @pl.when(c >= 1)
                def _():
                    drain(c - 1, o_rows, o_si)
                    write(c - 1, o_rows, o_so)

            return carry

        lax.fori_loop(0, n_steps // 2, body, 0)

        last = n_steps - 1
        rows, si, so = bufs[last % 2]
        o_rows, o_si, o_so = bufs[1 - last % 2]
        drain(last, rows, si)
        write(last, rows, so)
        wait_write(last - 1, o_rows, o_so)
        wait_write(last, rows, so)

    return gather_kernel


_gather = _make_gather()


def kernel(item_ids, table):
    ids = item_ids.astype(jnp.int32)
    idx2 = jnp.where(ids < _NMAIN, 2 * ids, 2 * (ids - _NMAIN) + 1)
    tail_t = table.T[:, _NMAIN:]
    tpad = _tp(table.T, tail_t)
    t2 = tpad.reshape(2 * _NMAIN, _HIDDEN)
    outp = _gather(idx2, t2)
    return outp[..., :_HIDDEN]
